# baseline (speedup 1.0000x reference)
import functools

import jax
import jax.numpy as jnp
from jax.experimental import pallas as pl

_TILE_N = 1024


def _upsample_kernel(xyz_ref, sx_ref, sf_ref, out_ref, *, S):
    sxc = [sx_ref[0, c : c + 1, :].reshape(S, 1) for c in range(3)]
    sfc = [sf_ref[0, c : c + 1, :].reshape(S, 1) for c in range(3)]
    d2 = (
        (sxc[0] - xyz_ref[0, 0:1, :]) ** 2
        + (sxc[1] - xyz_ref[0, 1:2, :]) ** 2
        + (sxc[2] - xyz_ref[0, 2:3, :]) ** 2
    )

    iota = jax.lax.broadcasted_iota(jnp.int32, d2.shape, 0)
    wsum = jnp.zeros((1, d2.shape[1]), jnp.float32)
    wmat = jnp.zeros(d2.shape, jnp.float32)
    d = d2
    for k in range(3):
        mk = jnp.min(d, axis=0, keepdims=True)
        first = jnp.argmin(d, axis=0, keepdims=True)
        hit = iota == first
        w = 1.0 / jnp.maximum(jnp.sqrt(mk), 1e-10)
        wmat = jnp.where(hit, w, wmat)
        wsum = wsum + w
        if k < 2:
            d = jnp.where(hit, jnp.inf, d)

    for c in range(3):
        f = jnp.sum(wmat * sfc[c], axis=0, keepdims=True)
        out_ref[0, c : c + 1, :] = jnp.clip(f / wsum, -100.0, 100.0)


def kernel(xyz, sparse_xyz, sparse_flow):
    B, C, N = xyz.shape
    S = sparse_xyz.shape[2]
    nt = N // _TILE_N

    return pl.pallas_call(
        functools.partial(_upsample_kernel, S=S),
        grid=(B, nt),
        in_specs=[
            pl.BlockSpec((1, C, _TILE_N), lambda b, t: (b, 0, t)),
            pl.BlockSpec((1, C, S), lambda b, t: (b, 0, 0)),
            pl.BlockSpec((1, C, S), lambda b, t: (b, 0, 0)),
        ],
        out_specs=pl.BlockSpec((1, C, _TILE_N), lambda b, t: (b, 0, t)),
        out_shape=jax.ShapeDtypeStruct((B, C, N), jnp.float32),
    )(xyz, sparse_xyz, sparse_flow)
